# baseline (speedup 1.0000x reference)
import functools

import jax
import jax.numpy as jnp
from jax import lax
from jax.experimental import pallas as pl
from jax.experimental.pallas import tpu as pltpu
from jax.experimental.pallas import tpu_sc as plsc

N = 10000
D = 128
E = 320000
L1 = 64
L2 = 32
WA = 80
WB = 64
CNT = 48
WC = 48

_NT = 16
ACC_ROWS = 10112
ROWS_PER_TILE = ACC_ROWS // _NT
DUMMY = N
E_PER_TILE = E // _NT
CHUNK = 80
K_FIRE = 5
NBB = E_PER_TILE // (2 * K_FIRE * CHUNK)
EROWS = E // CHUNK
TILE_EROWS = E_PER_TILE // CHUNK


def _norm_rows(v):
    n = jnp.sqrt(jnp.sum(v * v, axis=1, keepdims=True))
    return v / jnp.maximum(n, 1e-12)


def _make_segsum_body(w):
    def body(f_hbm, z_hbm, rp_hbm, cp_hbm, rn_hbm, cn_hbm, out_hbm,
             src2, dst2, rows2, acc_sh, gsem0, gsem1, ssem0, ssem1):
        c = lax.axis_index("c")
        s = lax.axis_index("s")
        row0 = s * ROWS_PER_TILE
        pltpu.sync_copy(z_hbm, acc_sh.at[pl.ds(row0, ROWS_PER_TILE)])
        plsc.subcore_barrier()

        gsem = (gsem0, gsem1)
        ssem = (ssem0, ssem1)
        base_row = s * TILE_EROWS

        def load_and_fire(idx_row, p):
            @pl.when(c == 0)
            def _():
                pltpu.sync_copy(rp_hbm.at[pl.ds(idx_row, K_FIRE)], dst2.at[p])
                pltpu.sync_copy(cp_hbm.at[pl.ds(idx_row, K_FIRE)], src2.at[p])

            @pl.when(c != 0)
            def _():
                pltpu.sync_copy(rn_hbm.at[pl.ds(idx_row, K_FIRE)], dst2.at[p])
                pltpu.sync_copy(cn_hbm.at[pl.ds(idx_row, K_FIRE)], src2.at[p])

            for r in range(K_FIRE):
                for j in range(CHUNK // 16):
                    sl = pl.ds(j * 16, 16)
                    v = dst2[p, r, sl]
                    dst2[p, r, sl] = jnp.where(v == src2[p, r, sl], DUMMY, v)
            for r in range(K_FIRE):
                pltpu.async_copy(f_hbm.at[src2.at[p, r]], rows2.at[p, r],
                                 gsem[p])

        def drain_gathers(p):
            for r in range(K_FIRE):
                pltpu.make_async_copy(f_hbm.at[src2.at[p, r]],
                                      rows2.at[p, r], gsem[p]).wait()

        def fire_scatters(p):
            for r in range(K_FIRE):
                pltpu.async_copy(rows2.at[p, r], acc_sh.at[dst2.at[p, r]],
                                 ssem[p], add=True)

        def drain_scatters(p):
            for r in range(K_FIRE):
                pltpu.make_async_copy(rows2.at[p, r],
                                      acc_sh.at[dst2.at[p, r]],
                                      ssem[p]).wait()

        load_and_fire(base_row, 0)

        def step(bb, _):
            row_a = base_row + (2 * bb) * K_FIRE
            drain_gathers(0)
            fire_scatters(0)

            @pl.when(bb > 0)
            def _():
                drain_scatters(1)

            load_and_fire(row_a + K_FIRE, 1)
            drain_gathers(1)
            fire_scatters(1)
            drain_scatters(0)

            @pl.when(bb < NBB - 1)
            def _():
                load_and_fire(row_a + 2 * K_FIRE, 0)

            return ()

        lax.fori_loop(0, NBB, step, (), unroll=False)
        drain_scatters(1)
        plsc.subcore_barrier()

        @pl.when(c == 0)
        def _():
            pltpu.sync_copy(acc_sh.at[pl.ds(row0, ROWS_PER_TILE)],
                            out_hbm.at[0, pl.ds(row0, ROWS_PER_TILE)])

        @pl.when(c != 0)
        def _():
            pltpu.sync_copy(acc_sh.at[pl.ds(row0, ROWS_PER_TILE)],
                            out_hbm.at[1, pl.ds(row0, ROWS_PER_TILE)])

    return body


@functools.cache
def _build_segsum(w):
    return pl.kernel(
        _make_segsum_body(w),
        out_type=jax.ShapeDtypeStruct((2, ACC_ROWS, w), jnp.float32),
        mesh=plsc.VectorSubcoreMesh(core_axis_name="c", subcore_axis_name="s",
                                    num_cores=2, num_subcores=_NT),
        scratch_types=[
            pltpu.VMEM((2, K_FIRE, CHUNK), jnp.int32),
            pltpu.VMEM((2, K_FIRE, CHUNK), jnp.int32),
            pltpu.VMEM((2, K_FIRE, CHUNK, w), jnp.float32),
            pltpu.VMEM_SHARED((ACC_ROWS, w), jnp.float32),
            pltpu.SemaphoreType.DMA,
            pltpu.SemaphoreType.DMA,
            pltpu.SemaphoreType.DMA,
            pltpu.SemaphoreType.DMA,
        ],
        compiler_params=pltpu.CompilerParams(use_tc_tiling_on_sc=False),
    )


def _segsum(w, *args):
    return _build_segsum(w)(*args)


_BM = 2000


def _lin0_body(x_ref, w_ref, b_ref, oa_ref, ob_ref):
    h = lax.dot_general(x_ref[...], w_ref[...], (((1,), (1,)), ((), ())),
                        preferred_element_type=jnp.float32) + b_ref[...]
    h = jnp.maximum(h, 0.0)
    oa_ref[...] = h[:, :WA]
    ob_ref[...] = jnp.concatenate(
        [h[:, WA:D], jnp.ones((h.shape[0], WB - CNT), jnp.float32)], axis=1)


def _base_body(sap_ref, san_ref, sbp_ref, sbn_ref, fa_ref, fb_ref,
               wp_ref, bp_ref, wn_ref, bn_ref, oa_ref, ob_ref):
    sap = sap_ref[0]
    san = san_ref[0]
    sbp = sbp_ref[0]
    sbn = sbn_ref[0]
    h = jnp.concatenate([fa_ref[...], fb_ref[...][:, :D - WA]], axis=1)
    sump = jnp.concatenate([sap, sbp[:, :D - WA]], axis=1)
    sumn = jnp.concatenate([san, sbn[:, :D - WA]], axis=1)
    aggp = sump / jnp.maximum(sbp[:, CNT:CNT + 1], 1.0)
    aggn = sumn / jnp.maximum(sbn[:, CNT:CNT + 1], 1.0)
    tp = jnp.tanh(_norm_rows(
        lax.dot_general(jnp.concatenate([aggp, h], axis=1), wp_ref[...],
                        (((1,), (0,)), ((), ())),
                        preferred_element_type=jnp.float32) + bp_ref[...]))
    tn = jnp.tanh(_norm_rows(
        lax.dot_general(jnp.concatenate([aggn, h], axis=1), wn_ref[...],
                        (((1,), (0,)), ((), ())),
                        preferred_element_type=jnp.float32) + bn_ref[...]))
    g = jnp.concatenate([tp, tn], axis=1)
    oa_ref[...] = g[:, :WA]
    ob_ref[...] = g[:, WA:]


def _deep_body(sap_ref, san_ref, sbp_ref, sbn_ref, cbp_ref, cbn_ref,
               ga_ref, gb_ref, wp_ref, bp_ref, wn_ref, bn_ref, o_ref):
    sap = sap_ref[0]
    san = san_ref[0]
    sbp = sbp_ref[0]
    sbn = sbn_ref[0]
    g = jnp.concatenate([ga_ref[...], gb_ref[...]], axis=1)
    hp0 = g[:, :L1]
    hn0 = g[:, L1:2 * L1]
    sump = jnp.concatenate([sap, sbp], axis=1)
    sumn = jnp.concatenate([san, sbn], axis=1)
    cntp = cbp_ref[0][:, CNT:CNT + 1] + 1.0
    cntn = cbn_ref[0][:, CNT:CNT + 1] + 1.0
    p_hp = (sump[:, :L1] + hp0) / cntp
    p_hn = (sump[:, L1:] + hn0) / cntp
    n_hn = (sumn[:, L1:] + hn0) / cntn
    n_hp = (sumn[:, :L1] + hp0) / cntn
    hp1 = jnp.tanh(_norm_rows(
        lax.dot_general(jnp.concatenate([p_hp, n_hn, hp0], axis=1),
                        wp_ref[...], (((1,), (0,)), ((), ())),
                        preferred_element_type=jnp.float32) + bp_ref[...]))
    hn1 = jnp.tanh(_norm_rows(
        lax.dot_general(jnp.concatenate([p_hn, n_hp, hn0], axis=1),
                        wn_ref[...], (((1,), (0,)), ((), ())),
                        preferred_element_type=jnp.float32) + bn_ref[...]))
    o_ref[...] = _norm_rows(jnp.concatenate([hp1, hn1], axis=1))


_BF = 80


def _final_body(a_ref, b_ref, m_ref, l_ref, p_ref, loss_ref):
    p = lax.dot_general(a_ref[...], b_ref[...], (((1,), (1,)), ((), ())),
                        preferred_element_type=jnp.float32) * m_ref[...]
    p_ref[...] = p
    d = p - l_ref[...]
    loss_ref[0, 0, 0] = jnp.sum(d * d)


def _sspec(w):
    return [pl.BlockSpec((1, _BM, w), lambda i: (0, i, 0)),
            pl.BlockSpec((1, _BM, w), lambda i: (1, i, 0))]


def kernel(X, positive_edges, negative_edges, labels, label_mask,
           W_lin, b_lin, W_pos_base, b_pos_base, W_neg_base, b_neg_base,
           W_pos_deep, b_pos_deep, W_neg_deep, b_neg_deep):
    za = jnp.zeros((ROWS_PER_TILE, WA), jnp.float32)
    zb = jnp.zeros((ROWS_PER_TILE, WB), jnp.float32)
    zc = jnp.zeros((ROWS_PER_TILE, WC), jnp.float32)
    rp2 = positive_edges[0].reshape(EROWS, CHUNK)
    cp2 = positive_edges[1].reshape(EROWS, CHUNK)
    rn2 = negative_edges[0].reshape(EROWS, CHUNK)
    cn2 = negative_edges[1].reshape(EROWS, CHUNK)

    Fa, Fb = pl.pallas_call(
        _lin0_body,
        grid=(N // _BM,),
        in_specs=[
            pl.BlockSpec((_BM, D), lambda i: (i, 0)),
            pl.BlockSpec((D, D), lambda i: (0, 0)),
            pl.BlockSpec((1, D), lambda i: (0, 0)),
        ],
        out_specs=[pl.BlockSpec((_BM, WA), lambda i: (i, 0)),
                   pl.BlockSpec((_BM, WB), lambda i: (i, 0))],
        out_shape=[jax.ShapeDtypeStruct((N, WA), jnp.float32),
                   jax.ShapeDtypeStruct((N, WB), jnp.float32)],
    )(X, W_lin, b_lin.reshape(1, D))

    SAb = _segsum(WA, Fa, za, rp2, cp2, rn2, cn2)
    SBb = _segsum(WB, Fb, zb, rp2, cp2, rn2, cn2)

    Ga, Gb = pl.pallas_call(
        _base_body,
        grid=(N // _BM,),
        in_specs=_sspec(WA) + _sspec(WB) + [
            pl.BlockSpec((_BM, WA), lambda i: (i, 0)),
            pl.BlockSpec((_BM, WB), lambda i: (i, 0)),
            pl.BlockSpec((2 * D, L1), lambda i: (0, 0)),
            pl.BlockSpec((1, L1), lambda i: (0, 0)),
            pl.BlockSpec((2 * D, L1), lambda i: (0, 0)),
            pl.BlockSpec((1, L1), lambda i: (0, 0)),
        ],
        out_specs=[pl.BlockSpec((_BM, WA), lambda i: (i, 0)),
                   pl.BlockSpec((_BM, WC), lambda i: (i, 0))],
        out_shape=[jax.ShapeDtypeStruct((N, WA), jnp.float32),
                   jax.ShapeDtypeStruct((N, WC), jnp.float32)],
    )(SAb, SAb, SBb, SBb, Fa, Fb, W_pos_base, b_pos_base.reshape(1, L1),
      W_neg_base, b_neg_base.reshape(1, L1))

    SAd = _segsum(WA, Ga, za, rp2, cp2, rn2, cn2)
    SBd = _segsum(WC, Gb, zc, rp2, cp2, rn2, cn2)

    X_mol = pl.pallas_call(
        _deep_body,
        grid=(N // _BM,),
        in_specs=_sspec(WA) + _sspec(WC) + _sspec(WB) + [
            pl.BlockSpec((_BM, WA), lambda i: (i, 0)),
            pl.BlockSpec((_BM, WC), lambda i: (i, 0)),
            pl.BlockSpec((3 * L1, L2), lambda i: (0, 0)),
            pl.BlockSpec((1, L2), lambda i: (0, 0)),
            pl.BlockSpec((3 * L1, L2), lambda i: (0, 0)),
            pl.BlockSpec((1, L2), lambda i: (0, 0)),
        ],
        out_specs=pl.BlockSpec((_BM, 2 * L2), lambda i: (i, 0)),
        out_shape=jax.ShapeDtypeStruct((N, 2 * L2), jnp.float32),
    )(SAd, SAd, SBd, SBd, SBb, SBb, Ga, Gb,
      W_pos_deep, b_pos_deep.reshape(1, L2),
      W_neg_deep, b_neg_deep.reshape(1, L2))

    gm = N // _BF
    pred2d, partials = pl.pallas_call(
        _final_body,
        grid=(gm,),
        in_specs=[
            pl.BlockSpec((_BF, 2 * L2), lambda i: (i, 0)),
            pl.BlockSpec((N, 2 * L2), lambda i: (0, 0)),
            pl.BlockSpec((_BF, N), lambda i: (i, 0)),
            pl.BlockSpec((_BF, N), lambda i: (i, 0)),
        ],
        out_specs=[
            pl.BlockSpec((_BF, N), lambda i: (i, 0)),
            pl.BlockSpec((1, 1, 1), lambda i: (i, 0, 0),
                         memory_space=pltpu.SMEM),
        ],
        out_shape=[
            jax.ShapeDtypeStruct((N, N), jnp.float32),
            jax.ShapeDtypeStruct((gm, 1, 1), jnp.float32),
        ],
    )(X_mol, X_mol, label_mask, labels.reshape(N, N))

    loss = jnp.sum(partials) / (N * N)
    return (loss, X_mol, pred2d.reshape(-1))
